# Initial kernel scaffold; baseline (speedup 1.0000x reference)
import jax
import jax.numpy as jnp
from jax.experimental import pallas as pl


def kernel(x, edge_index, W1, b1, gn1_w, gn1_b, gn1_a, W2, b2, gn2_w, gn2_b, gn2_a, l1_w, l1_b, l2_w, l2_b, l3_w, l3_b, l4_w, l4_b):
    raise NotImplementedError("write your pallas kernel here")


# speedup vs baseline: 25.4408x; 25.4408x over previous
import functools

import jax
import jax.numpy as jnp
from jax import lax
from jax.experimental import pallas as pl
from jax.experimental.pallas import tpu as pltpu
from jax.experimental.pallas import tpu_sc as plsc

N = 100000
E = 1600000
NC = 2
NS = 16
NW = NC * NS
RPT = 6272
NPAD = NS * RPT
ZCH = 1568
K = 2000

BLK = 4000
GRID = N // BLK

_MESH = dict(core_axis_name="c", subcore_axis_name="s")


def _sc_degree(edge_index, zeros_rpt, ones_k):
    epw = E // NW
    steps = epw // K

    @functools.partial(
        pl.kernel,
        out_type=jax.ShapeDtypeStruct((NC, NPAD), jnp.float32),
        mesh=plsc.VectorSubcoreMesh(**_MESH),
        scratch_types=[
            pltpu.VMEM((K,), jnp.int32),
            pltpu.VMEM((K,), jnp.float32),
            pltpu.VMEM_SHARED((NPAD,), jnp.float32),
        ],
    )
    def k(edge_hbm, z_hbm, ones_hbm, out_hbm, idx_v, ones_v, acc):
        c = lax.axis_index("c")
        s = lax.axis_index("s")
        w = s * NC + c
        pltpu.sync_copy(ones_hbm, ones_v)
        pltpu.sync_copy(z_hbm, acc.at[pl.ds(s * RPT, RPT)])
        plsc.subcore_barrier()
        ebase = w * epw

        def body(i, carry):
            pltpu.sync_copy(edge_hbm.at[pl.ds(E + ebase + i * K, K)], idx_v)
            pltpu.sync_copy(ones_v, acc.at[idx_v], add=True)
            return carry

        lax.fori_loop(0, steps, body, 0)
        plsc.subcore_barrier()
        pltpu.sync_copy(acc.at[pl.ds(s * RPT, RPT)],
                        out_hbm.at[c, pl.ds(s * RPT, RPT)])

    return k(edge_index, zeros_rpt, ones_k)


def _sc_conv1(edge_index, u1, zeros_z):
    epw = E // NW
    steps = epw // K

    @functools.partial(
        pl.kernel,
        out_type=jax.ShapeDtypeStruct((NC, NPAD, 8), jnp.float32),
        mesh=plsc.VectorSubcoreMesh(**_MESH),
        compiler_params=pltpu.CompilerParams(use_tc_tiling_on_sc=False),
        scratch_types=[
            pltpu.VMEM((K,), jnp.int32),
            pltpu.VMEM((K,), jnp.int32),
            pltpu.VMEM((K, 8), jnp.float32),
            pltpu.VMEM_SHARED((NPAD, 8), jnp.float32),
            pltpu.SemaphoreType.DMA,
        ],
    )
    def k(edge_hbm, u_hbm, z_hbm, out_hbm, src_v, dst_v, rows_v, acc, sem):
        c = lax.axis_index("c")
        s = lax.axis_index("s")
        w = s * NC + c
        for j in range(RPT // ZCH):
            pltpu.sync_copy(z_hbm, acc.at[pl.ds(s * RPT + j * ZCH, ZCH)])
        plsc.subcore_barrier()
        ebase = w * epw

        def body(i, carry):
            base = ebase + i * K
            pltpu.sync_copy(edge_hbm.at[pl.ds(base, K)], src_v)
            pltpu.sync_copy(edge_hbm.at[pl.ds(E + base, K)], dst_v)
            pltpu.async_copy(u_hbm.at[src_v], rows_v, sem).wait()
            pltpu.sync_copy(rows_v, acc.at[dst_v], add=True)
            return carry

        lax.fori_loop(0, steps, body, 0)
        plsc.subcore_barrier()
        pltpu.sync_copy(acc.at[pl.ds(s * RPT, RPT)],
                        out_hbm.at[c, pl.ds(s * RPT, RPT)])

    return k(edge_index, u1, zeros_z)


def _sc_conv2(edge_index, ua, ub, zeros_z):
    K2 = 1000
    eps_ = E // NS
    steps = eps_ // K2

    @functools.partial(
        pl.kernel,
        out_type=jax.ShapeDtypeStruct((NC, NPAD, 16), jnp.float32),
        mesh=plsc.VectorSubcoreMesh(**_MESH),
        compiler_params=pltpu.CompilerParams(use_tc_tiling_on_sc=False),
        scratch_types=[
            pltpu.VMEM((K2,), jnp.int32),
            pltpu.VMEM((K2,), jnp.int32),
            pltpu.VMEM((K2, 16), jnp.float32),
            pltpu.VMEM_SHARED((NPAD, 16), jnp.float32),
            pltpu.SemaphoreType.DMA,
        ],
    )
    def k(edge_hbm, ua_hbm, ub_hbm, z_hbm, out_hbm, src_v, dst_v, rows_v, acc, sem):
        c = lax.axis_index("c")
        s = lax.axis_index("s")
        for j in range(RPT // ZCH):
            pltpu.sync_copy(z_hbm, acc.at[pl.ds(s * RPT + j * ZCH, ZCH)])
        plsc.subcore_barrier()
        ebase = s * eps_

        def run(u_hbm):
            def body(i, carry):
                base = ebase + i * K2
                pltpu.sync_copy(edge_hbm.at[pl.ds(base, K2)], src_v)
                pltpu.sync_copy(edge_hbm.at[pl.ds(E + base, K2)], dst_v)
                pltpu.async_copy(u_hbm.at[src_v], rows_v, sem).wait()
                pltpu.sync_copy(rows_v, acc.at[dst_v], add=True)
                return carry
            lax.fori_loop(0, steps, body, 0)

        @pl.when(c == 0)
        def _():
            run(ua_hbm)

        @pl.when(c == 1)
        def _():
            run(ub_hbm)

        plsc.subcore_barrier()
        pltpu.sync_copy(acc.at[pl.ds(s * RPT, RPT)],
                        out_hbm.at[c, pl.ds(s * RPT, RPT)])

    return k(edge_index, ua, ub, zeros_z)


def _row_spec(f):
    return pl.BlockSpec((BLK, f), lambda i: (i, 0))


def _full_spec(shape):
    return pl.BlockSpec(shape, lambda i: tuple(0 for _ in shape))


def _silu(z):
    return z * jax.nn.sigmoid(z)


def _tc_a(x, da, db):
    def body(x_ref, da_ref, db_ref, u1_ref, dinv_ref):
        dinv = lax.rsqrt(da_ref[...] + db_ref[...] + 1.0)
        dinv_ref[...] = dinv
        u1_ref[...] = x_ref[...] * dinv

    return pl.pallas_call(
        body,
        grid=(GRID,),
        in_specs=[_row_spec(8), _row_spec(1), _row_spec(1)],
        out_specs=[_row_spec(8), _row_spec(1)],
        out_shape=[jax.ShapeDtypeStruct((N, 8), jnp.float32),
                   jax.ShapeDtypeStruct((N, 1), jnp.float32)],
    )(x, da, db)


def _tc_b(t1a, t1b, u1, dinv, W1, b1):
    def body(t1a_ref, t1b_ref, u1_ref, dinv_ref, W_ref, b_ref, h_ref, S_ref, Q_ref):
        pid = pl.program_id(0)
        agg = dinv_ref[...] * (t1a_ref[...] + t1b_ref[...] + u1_ref[...])
        z = jnp.dot(agg, W_ref[...], preferred_element_type=jnp.float32, precision=lax.Precision.HIGHEST) + b_ref[...]
        h = _silu(z)
        h_ref[...] = h

        @pl.when(pid == 0)
        def _():
            S_ref[...] = jnp.zeros_like(S_ref)
            Q_ref[...] = jnp.zeros_like(Q_ref)

        S_ref[...] += jnp.sum(h, axis=0, keepdims=True)
        Q_ref[...] += jnp.sum(h * h, axis=0, keepdims=True)

    return pl.pallas_call(
        body,
        grid=(GRID,),
        in_specs=[_row_spec(8), _row_spec(8), _row_spec(8), _row_spec(1),
                  _full_spec((8, 32)), _full_spec((1, 32))],
        out_specs=[_row_spec(32), _full_spec((1, 32)), _full_spec((1, 32))],
        out_shape=[jax.ShapeDtypeStruct((N, 32), jnp.float32),
                   jax.ShapeDtypeStruct((1, 32), jnp.float32),
                   jax.ShapeDtypeStruct((1, 32), jnp.float32)],
    )(t1a, t1b, u1, dinv, W1, b1)


def _tc_c(h1, S1, Q1, dinv, gw, gb, ga):
    def body(h_ref, S_ref, Q_ref, dinv_ref, gw_ref, gb_ref, ga_ref, ua_ref, ub_ref):
        a = ga_ref[...]
        m = S_ref[...] * (1.0 / N)
        var = Q_ref[...] * (1.0 / N) - (2.0 * a - a * a) * m * m
        g = gw_ref[...] * (h_ref[...] - a * m) * lax.rsqrt(var + 1e-5) + gb_ref[...]
        u2 = dinv_ref[...] * g
        ua_ref[...] = u2[:, :16]
        ub_ref[...] = u2[:, 16:]

    return pl.pallas_call(
        body,
        grid=(GRID,),
        in_specs=[_row_spec(32), _full_spec((1, 32)), _full_spec((1, 32)),
                  _row_spec(1), _full_spec((1, 32)), _full_spec((1, 32)),
                  _full_spec((1, 32))],
        out_specs=[_row_spec(16), _row_spec(16)],
        out_shape=[jax.ShapeDtypeStruct((N, 16), jnp.float32),
                   jax.ShapeDtypeStruct((N, 16), jnp.float32)],
    )(h1, S1, Q1, dinv, gw, gb, ga)


def _tc_d(t2a, t2b, ua, ub, dinv, W2, b2):
    def body(t2a_ref, t2b_ref, ua_ref, ub_ref, dinv_ref, W_ref, b_ref,
             h_ref, S_ref, Q_ref):
        pid = pl.program_id(0)
        dinv = dinv_ref[...]
        agg = jnp.concatenate(
            [dinv * (t2a_ref[...] + ua_ref[...]),
             dinv * (t2b_ref[...] + ub_ref[...])], axis=1)
        z = jnp.dot(agg, W_ref[...], preferred_element_type=jnp.float32, precision=lax.Precision.HIGHEST) + b_ref[...]
        h = _silu(z)
        h_ref[...] = h

        @pl.when(pid == 0)
        def _():
            S_ref[...] = jnp.zeros_like(S_ref)
            Q_ref[...] = jnp.zeros_like(Q_ref)

        S_ref[...] += jnp.sum(h, axis=0, keepdims=True)
        Q_ref[...] += jnp.sum(h * h, axis=0, keepdims=True)

    return pl.pallas_call(
        body,
        grid=(GRID,),
        in_specs=[_row_spec(16), _row_spec(16), _row_spec(16), _row_spec(16),
                  _row_spec(1), _full_spec((32, 64)), _full_spec((1, 64))],
        out_specs=[_row_spec(64), _full_spec((1, 64)), _full_spec((1, 64))],
        out_shape=[jax.ShapeDtypeStruct((N, 64), jnp.float32),
                   jax.ShapeDtypeStruct((1, 64), jnp.float32),
                   jax.ShapeDtypeStruct((1, 64), jnp.float32)],
    )(t2a, t2b, ua, ub, dinv, W2, b2)


def _tc_e(h2, S2, Q2, gw, gb, ga, w1, c1, w2, c2, w3, c3, w4, c4):
    def body(h_ref, S_ref, Q_ref, gw_ref, gb_ref, ga_ref,
             w1_ref, c1_ref, w2_ref, c2_ref, w3_ref, c3_ref, w4_ref, c4_ref,
             out_ref):
        a = ga_ref[...]
        m = S_ref[...] * (1.0 / N)
        var = Q_ref[...] * (1.0 / N) - (2.0 * a - a * a) * m * m
        g = gw_ref[...] * (h_ref[...] - a * m) * lax.rsqrt(var + 1e-5) + gb_ref[...]
        y = _silu(jnp.dot(g, w1_ref[...], preferred_element_type=jnp.float32, precision=lax.Precision.HIGHEST) + c1_ref[...])
        y = _silu(jnp.dot(y, w2_ref[...], preferred_element_type=jnp.float32, precision=lax.Precision.HIGHEST) + c2_ref[...])
        y = _silu(jnp.dot(y, w3_ref[...], preferred_element_type=jnp.float32, precision=lax.Precision.HIGHEST) + c3_ref[...])
        out_ref[...] = jnp.dot(y, w4_ref[...], preferred_element_type=jnp.float32, precision=lax.Precision.HIGHEST) + c4_ref[...]

    return pl.pallas_call(
        body,
        grid=(GRID,),
        in_specs=[_row_spec(64), _full_spec((1, 64)), _full_spec((1, 64)),
                  _full_spec((1, 64)), _full_spec((1, 64)), _full_spec((1, 64)),
                  _full_spec((64, 64)), _full_spec((1, 64)),
                  _full_spec((64, 32)), _full_spec((1, 32)),
                  _full_spec((32, 16)), _full_spec((1, 16)),
                  _full_spec((16, 4)), _full_spec((1, 4))],
        out_specs=[_row_spec(4)],
        out_shape=[jax.ShapeDtypeStruct((N, 4), jnp.float32)],
    )(h2, S2, Q2, gw, gb, ga, w1, c1, w2, c2, w3, c3, w4, c4)[0]


def kernel(x, edge_index, W1, b1, gn1_w, gn1_b, gn1_a, W2, b2, gn2_w, gn2_b,
           gn2_a, l1_w, l1_b, l2_w, l2_b, l3_w, l3_b, l4_w, l4_b):
    eflat = edge_index.reshape(2 * E)
    zeros_rpt = jnp.zeros((RPT,), jnp.float32)
    ones_k = jnp.ones((K,), jnp.float32)
    zeros_z8 = jnp.zeros((ZCH, 8), jnp.float32)
    zeros_z16 = jnp.zeros((ZCH, 16), jnp.float32)

    degp = _sc_degree(eflat, zeros_rpt, ones_k)
    da = degp[0, :N].reshape(N, 1)
    db = degp[1, :N].reshape(N, 1)
    u1, dinv = _tc_a(x, da, db)

    t1p = _sc_conv1(eflat, u1, zeros_z8)
    t1a = t1p[0, :N]
    t1b = t1p[1, :N]
    h1, S1, Q1 = _tc_b(t1a, t1b, u1, dinv, W1, b1.reshape(1, 32))

    ua, ub = _tc_c(h1, S1, Q1, dinv, gn1_w.reshape(1, 32), gn1_b.reshape(1, 32),
                   gn1_a.reshape(1, 32))

    t2 = _sc_conv2(eflat, ua, ub, zeros_z16)
    t2a = t2[0, :N]
    t2b = t2[1, :N]
    h2, S2, Q2 = _tc_d(t2a, t2b, ua, ub, dinv, W2, b2.reshape(1, 64))

    return _tc_e(h2, S2, Q2, gn2_w.reshape(1, 64), gn2_b.reshape(1, 64),
                 gn2_a.reshape(1, 64), l1_w, l1_b.reshape(1, 64),
                 l2_w, l2_b.reshape(1, 32), l3_w, l3_b.reshape(1, 16),
                 l4_w, l4_b.reshape(1, 4))


# speedup vs baseline: 32.5463x; 1.2793x over previous
import functools

import jax
import jax.numpy as jnp
from jax import lax
from jax.experimental import pallas as pl
from jax.experimental.pallas import tpu as pltpu
from jax.experimental.pallas import tpu_sc as plsc

N = 100000
E = 1600000
NC = 2
NS = 16
NW = NC * NS
RPT = 6272
NPAD = NS * RPT
ZCH = 1568
K = 2000

BLK = 4000
GRID = N // BLK

_MESH = dict(core_axis_name="c", subcore_axis_name="s")


def _sc_degree(edge_index, zeros_z, ones_k):
    epw = E // NW
    steps = epw // K

    @functools.partial(
        pl.kernel,
        out_type=jax.ShapeDtypeStruct((NC, NPAD, 8), jnp.float32),
        mesh=plsc.VectorSubcoreMesh(**_MESH),
        compiler_params=pltpu.CompilerParams(use_tc_tiling_on_sc=False),
        scratch_types=[
            pltpu.VMEM((K,), jnp.int32),
            pltpu.VMEM((K, 8), jnp.float32),
            pltpu.VMEM_SHARED((NPAD, 8), jnp.float32),
        ],
    )
    def k(edge_hbm, z_hbm, ones_hbm, out_hbm, idx_v, ones_v, acc):
        c = lax.axis_index("c")
        s = lax.axis_index("s")
        w = s * NC + c
        pltpu.sync_copy(ones_hbm, ones_v)
        for j in range(RPT // ZCH):
            pltpu.sync_copy(z_hbm, acc.at[pl.ds(s * RPT + j * ZCH, ZCH)])
        plsc.subcore_barrier()
        ebase = w * epw

        def body(i, carry):
            pltpu.sync_copy(edge_hbm.at[pl.ds(E + ebase + i * K, K)], idx_v)
            pltpu.sync_copy(ones_v, acc.at[idx_v], add=True)
            return carry

        lax.fori_loop(0, steps, body, 0)
        plsc.subcore_barrier()
        pltpu.sync_copy(acc.at[pl.ds(s * RPT, RPT)],
                        out_hbm.at[c, pl.ds(s * RPT, RPT)])

    return k(edge_index, zeros_z, ones_k)


def _sc_conv1(edge_index, u1, zeros_z):
    epw = E // NW
    steps = epw // K

    @functools.partial(
        pl.kernel,
        out_type=jax.ShapeDtypeStruct((NC, NPAD, 8), jnp.float32),
        mesh=plsc.VectorSubcoreMesh(**_MESH),
        compiler_params=pltpu.CompilerParams(use_tc_tiling_on_sc=False),
        scratch_types=[
            pltpu.VMEM((K,), jnp.int32),
            pltpu.VMEM((K,), jnp.int32),
            pltpu.VMEM((K, 8), jnp.float32),
            pltpu.VMEM_SHARED((NPAD, 8), jnp.float32),
            pltpu.SemaphoreType.DMA,
        ],
    )
    def k(edge_hbm, u_hbm, z_hbm, out_hbm, src_v, dst_v, rows_v, acc, sem):
        c = lax.axis_index("c")
        s = lax.axis_index("s")
        w = s * NC + c
        for j in range(RPT // ZCH):
            pltpu.sync_copy(z_hbm, acc.at[pl.ds(s * RPT + j * ZCH, ZCH)])
        plsc.subcore_barrier()
        ebase = w * epw

        def body(i, carry):
            base = ebase + i * K
            pltpu.sync_copy(edge_hbm.at[pl.ds(base, K)], src_v)
            pltpu.sync_copy(edge_hbm.at[pl.ds(E + base, K)], dst_v)
            pltpu.async_copy(u_hbm.at[src_v], rows_v, sem).wait()
            pltpu.sync_copy(rows_v, acc.at[dst_v], add=True)
            return carry

        lax.fori_loop(0, steps, body, 0)
        plsc.subcore_barrier()
        pltpu.sync_copy(acc.at[pl.ds(s * RPT, RPT)],
                        out_hbm.at[c, pl.ds(s * RPT, RPT)])

    return k(edge_index, u1, zeros_z)


def _sc_conv2(edge_index, ua, ub, zeros_z):
    K2 = 1000
    eps_ = E // NS
    steps = eps_ // K2

    @functools.partial(
        pl.kernel,
        out_type=jax.ShapeDtypeStruct((NC, NPAD, 16), jnp.float32),
        mesh=plsc.VectorSubcoreMesh(**_MESH),
        compiler_params=pltpu.CompilerParams(use_tc_tiling_on_sc=False),
        scratch_types=[
            pltpu.VMEM((K2,), jnp.int32),
            pltpu.VMEM((K2,), jnp.int32),
            pltpu.VMEM((K2, 16), jnp.float32),
            pltpu.VMEM_SHARED((NPAD, 16), jnp.float32),
            pltpu.SemaphoreType.DMA,
        ],
    )
    def k(edge_hbm, ua_hbm, ub_hbm, z_hbm, out_hbm, src_v, dst_v, rows_v, acc, sem):
        c = lax.axis_index("c")
        s = lax.axis_index("s")
        for j in range(RPT // ZCH):
            pltpu.sync_copy(z_hbm, acc.at[pl.ds(s * RPT + j * ZCH, ZCH)])
        plsc.subcore_barrier()
        ebase = s * eps_

        def run(u_hbm):
            def body(i, carry):
                base = ebase + i * K2
                pltpu.sync_copy(edge_hbm.at[pl.ds(base, K2)], src_v)
                pltpu.sync_copy(edge_hbm.at[pl.ds(E + base, K2)], dst_v)
                pltpu.async_copy(u_hbm.at[src_v], rows_v, sem).wait()
                pltpu.sync_copy(rows_v, acc.at[dst_v], add=True)
                return carry
            lax.fori_loop(0, steps, body, 0)

        @pl.when(c == 0)
        def _():
            run(ua_hbm)

        @pl.when(c == 1)
        def _():
            run(ub_hbm)

        plsc.subcore_barrier()
        pltpu.sync_copy(acc.at[pl.ds(s * RPT, RPT)],
                        out_hbm.at[c, pl.ds(s * RPT, RPT)])

    return k(edge_index, ua, ub, zeros_z)


def _row_spec(f):
    return pl.BlockSpec((BLK, f), lambda i: (i, 0))


def _part_spec(core, f):
    return pl.BlockSpec((1, BLK, f), lambda i, _c=core: (_c, i, 0))


def _full_spec(shape):
    return pl.BlockSpec(shape, lambda i: tuple(0 for _ in shape))


def _silu(z):
    return z * jax.nn.sigmoid(z)


def _dot(a, b):
    return jnp.dot(a, b, preferred_element_type=jnp.float32)


def _sq(ref):
    return ref[0]


def _tc_a(x, degp):
    def body(x_ref, da_ref, db_ref, u1_ref, dinv_ref):
        dinv = lax.rsqrt(_sq(da_ref) + _sq(db_ref) + 1.0)
        dinv_ref[...] = dinv
        u1_ref[...] = x_ref[...] * dinv

    return pl.pallas_call(
        body,
        grid=(GRID,),
        in_specs=[_row_spec(8), _part_spec(0, 8), _part_spec(1, 8)],
        out_specs=[_row_spec(8), _row_spec(8)],
        out_shape=[jax.ShapeDtypeStruct((N, 8), jnp.float32),
                   jax.ShapeDtypeStruct((N, 8), jnp.float32)],
    )(x, degp, degp)


def _tc_b(t1p, u1, dinv8, W1, b1):
    def body(t1a_ref, t1b_ref, u1_ref, dinv_ref, W_ref, b_ref, h_ref, S_ref, Q_ref):
        pid = pl.program_id(0)
        agg = dinv_ref[...] * (_sq(t1a_ref) + _sq(t1b_ref) + u1_ref[...])
        z = _dot(agg, W_ref[...]) + b_ref[...]
        h = _silu(z)
        h_ref[...] = h

        @pl.when(pid == 0)
        def _():
            S_ref[...] = jnp.zeros_like(S_ref)
            Q_ref[...] = jnp.zeros_like(Q_ref)

        S_ref[...] += jnp.sum(h, axis=0, keepdims=True)
        Q_ref[...] += jnp.sum(h * h, axis=0, keepdims=True)

    return pl.pallas_call(
        body,
        grid=(GRID,),
        in_specs=[_part_spec(0, 8), _part_spec(1, 8), _row_spec(8), _row_spec(8),
                  _full_spec((8, 32)), _full_spec((1, 32))],
        out_specs=[_row_spec(32), _full_spec((1, 32)), _full_spec((1, 32))],
        out_shape=[jax.ShapeDtypeStruct((N, 32), jnp.float32),
                   jax.ShapeDtypeStruct((1, 32), jnp.float32),
                   jax.ShapeDtypeStruct((1, 32), jnp.float32)],
    )(t1p, t1p, u1, dinv8, W1, b1)


def _tc_c(h1, S1, Q1, dinv8, gw, gb, ga):
    def body(h_ref, S_ref, Q_ref, dinv_ref, gw_ref, gb_ref, ga_ref, ua_ref, ub_ref):
        a = ga_ref[...]
        m = S_ref[...] * (1.0 / N)
        var = Q_ref[...] * (1.0 / N) - (2.0 * a - a * a) * m * m
        g = gw_ref[...] * (h_ref[...] - a * m) * lax.rsqrt(var + 1e-5) + gb_ref[...]
        u2 = dinv_ref[:, :1] * g
        ua_ref[...] = u2[:, :16]
        ub_ref[...] = u2[:, 16:]

    return pl.pallas_call(
        body,
        grid=(GRID,),
        in_specs=[_row_spec(32), _full_spec((1, 32)), _full_spec((1, 32)),
                  _row_spec(8), _full_spec((1, 32)), _full_spec((1, 32)),
                  _full_spec((1, 32))],
        out_specs=[_row_spec(16), _row_spec(16)],
        out_shape=[jax.ShapeDtypeStruct((N, 16), jnp.float32),
                   jax.ShapeDtypeStruct((N, 16), jnp.float32)],
    )(h1, S1, Q1, dinv8, gw, gb, ga)


def _tc_d(t2, ua, ub, dinv8, W2, b2):
    def body(t2a_ref, t2b_ref, ua_ref, ub_ref, dinv_ref, W_ref, b_ref,
             h_ref, S_ref, Q_ref):
        pid = pl.program_id(0)
        dinv = dinv_ref[:, :1]
        agg = jnp.concatenate(
            [dinv * (_sq(t2a_ref) + ua_ref[...]),
             dinv * (_sq(t2b_ref) + ub_ref[...])], axis=1)
        z = _dot(agg, W_ref[...]) + b_ref[...]
        h = _silu(z)
        h_ref[...] = h

        @pl.when(pid == 0)
        def _():
            S_ref[...] = jnp.zeros_like(S_ref)
            Q_ref[...] = jnp.zeros_like(Q_ref)

        S_ref[...] += jnp.sum(h, axis=0, keepdims=True)
        Q_ref[...] += jnp.sum(h * h, axis=0, keepdims=True)

    return pl.pallas_call(
        body,
        grid=(GRID,),
        in_specs=[_part_spec(0, 16), _part_spec(1, 16), _row_spec(16),
                  _row_spec(16), _row_spec(8),
                  _full_spec((32, 64)), _full_spec((1, 64))],
        out_specs=[_row_spec(64), _full_spec((1, 64)), _full_spec((1, 64))],
        out_shape=[jax.ShapeDtypeStruct((N, 64), jnp.float32),
                   jax.ShapeDtypeStruct((1, 64), jnp.float32),
                   jax.ShapeDtypeStruct((1, 64), jnp.float32)],
    )(t2, t2, ua, ub, dinv8, W2, b2)


def _tc_e(h2, S2, Q2, gw, gb, ga, w1, c1, w2, c2, w3, c3, w4, c4):
    def body(h_ref, S_ref, Q_ref, gw_ref, gb_ref, ga_ref,
             w1_ref, c1_ref, w2_ref, c2_ref, w3_ref, c3_ref, w4_ref, c4_ref,
             out_ref):
        a = ga_ref[...]
        m = S_ref[...] * (1.0 / N)
        var = Q_ref[...] * (1.0 / N) - (2.0 * a - a * a) * m * m
        g = gw_ref[...] * (h_ref[...] - a * m) * lax.rsqrt(var + 1e-5) + gb_ref[...]
        y = _silu(_dot(g, w1_ref[...]) + c1_ref[...])
        y = _silu(_dot(y, w2_ref[...]) + c2_ref[...])
        y = _silu(_dot(y, w3_ref[...]) + c3_ref[...])
        out_ref[...] = _dot(y, w4_ref[...]) + c4_ref[...]

    return pl.pallas_call(
        body,
        grid=(GRID,),
        in_specs=[_row_spec(64), _full_spec((1, 64)), _full_spec((1, 64)),
                  _full_spec((1, 64)), _full_spec((1, 64)), _full_spec((1, 64)),
                  _full_spec((64, 64)), _full_spec((1, 64)),
                  _full_spec((64, 32)), _full_spec((1, 32)),
                  _full_spec((32, 16)), _full_spec((1, 16)),
                  _full_spec((16, 4)), _full_spec((1, 4))],
        out_specs=[_row_spec(4)],
        out_shape=[jax.ShapeDtypeStruct((N, 4), jnp.float32)],
    )(h2, S2, Q2, gw, gb, ga, w1, c1, w2, c2, w3, c3, w4, c4)[0]


def kernel(x, edge_index, W1, b1, gn1_w, gn1_b, gn1_a, W2, b2, gn2_w, gn2_b,
           gn2_a, l1_w, l1_b, l2_w, l2_b, l3_w, l3_b, l4_w, l4_b):
    eflat = edge_index.reshape(2 * E)
    ones_k8 = jnp.ones((K, 8), jnp.float32)
    zeros_z8 = jnp.zeros((ZCH, 8), jnp.float32)
    zeros_z16 = jnp.zeros((ZCH, 16), jnp.float32)

    degp = _sc_degree(eflat, zeros_z8, ones_k8)
    u1, dinv8 = _tc_a(x, degp)

    t1p = _sc_conv1(eflat, u1, zeros_z8)
    h1, S1, Q1 = _tc_b(t1p, u1, dinv8, W1, b1.reshape(1, 32))

    ua, ub = _tc_c(h1, S1, Q1, dinv8, gn1_w.reshape(1, 32), gn1_b.reshape(1, 32),
                   gn1_a.reshape(1, 32))

    t2 = _sc_conv2(eflat, ua, ub, zeros_z16)
    h2, S2, Q2 = _tc_d(t2, ua, ub, dinv8, W2, b2.reshape(1, 64))

    return _tc_e(h2, S2, Q2, gn2_w.reshape(1, 64), gn2_b.reshape(1, 64),
                 gn2_a.reshape(1, 64), l1_w, l1_b.reshape(1, 64),
                 l2_w, l2_b.reshape(1, 32), l3_w, l3_b.reshape(1, 16),
                 l4_w, l4_b.reshape(1, 4))


# speedup vs baseline: 44.2193x; 1.3587x over previous
import functools

import jax
import jax.numpy as jnp
from jax import lax
from jax.experimental import pallas as pl
from jax.experimental.pallas import tpu as pltpu
from jax.experimental.pallas import tpu_sc as plsc

N = 100000
E = 1600000
NC = 2
NS = 16
NW = NC * NS
RPT = 6400
NPAD = NS * RPT
ZCH = 1600
K = 1000

PD = NPAD * 16 // 128
PROW = N * 16 // 128
BP = 512
GRID = PD // BP

_MESH = dict(core_axis_name="c", subcore_axis_name="s")


def _sc_degree(eflat, zeros_z, ones_k):
    epw = E // NW
    steps = epw // K

    @functools.partial(
        pl.kernel,
        out_type=jax.ShapeDtypeStruct((NC, NPAD, 16), jnp.float32),
        mesh=plsc.VectorSubcoreMesh(**_MESH),
        compiler_params=pltpu.CompilerParams(use_tc_tiling_on_sc=False),
        scratch_types=[
            pltpu.VMEM((K,), jnp.int32),
            pltpu.VMEM((K, 16), jnp.float32),
            pltpu.VMEM_SHARED((NPAD, 16), jnp.float32),
        ],
    )
    def k(edge_hbm, z_hbm, ones_hbm, out_hbm, idx_v, ones_v, acc):
        c = lax.axis_index("c")
        s = lax.axis_index("s")
        w = s * NC + c
        pltpu.sync_copy(ones_hbm, ones_v)
        for j in range(RPT // ZCH):
            pltpu.sync_copy(z_hbm, acc.at[pl.ds(s * RPT + j * ZCH, ZCH)])
        plsc.subcore_barrier()
        ebase = w * epw

        def body(i, carry):
            pltpu.sync_copy(edge_hbm.at[pl.ds(E + ebase + i * K, K)], idx_v)
            pltpu.sync_copy(ones_v, acc.at[idx_v], add=True)
            return carry

        lax.fori_loop(0, steps, body, 0)
        plsc.subcore_barrier()
        pltpu.sync_copy(acc.at[pl.ds(s * RPT, RPT)],
                        out_hbm.at[c, pl.ds(s * RPT, RPT)])

    return k(eflat, zeros_z, ones_k)


def _sc_conv1(eflat, u1, zeros_z):
    epw = E // NW
    steps = epw // K

    @functools.partial(
        pl.kernel,
        out_type=jax.ShapeDtypeStruct((NC, NPAD, 16), jnp.float32),
        mesh=plsc.VectorSubcoreMesh(**_MESH),
        compiler_params=pltpu.CompilerParams(use_tc_tiling_on_sc=False),
        scratch_types=[
            pltpu.VMEM((K,), jnp.int32),
            pltpu.VMEM((K,), jnp.int32),
            pltpu.VMEM((K, 16), jnp.float32),
            pltpu.VMEM_SHARED((NPAD, 16), jnp.float32),
            pltpu.SemaphoreType.DMA,
        ],
    )
    def k(edge_hbm, u_hbm, z_hbm, out_hbm, src_v, dst_v, rows_v, acc, sem):
        c = lax.axis_index("c")
        s = lax.axis_index("s")
        w = s * NC + c
        for j in range(RPT // ZCH):
            pltpu.sync_copy(z_hbm, acc.at[pl.ds(s * RPT + j * ZCH, ZCH)])
        plsc.subcore_barrier()
        ebase = w * epw

        def body(i, carry):
            base = ebase + i * K
            pltpu.sync_copy(edge_hbm.at[pl.ds(base, K)], src_v)
            pltpu.sync_copy(edge_hbm.at[pl.ds(E + base, K)], dst_v)
            pltpu.async_copy(u_hbm.at[src_v], rows_v, sem).wait()
            pltpu.sync_copy(rows_v, acc.at[dst_v], add=True)
            return carry

        lax.fori_loop(0, steps, body, 0)
        plsc.subcore_barrier()
        pltpu.sync_copy(acc.at[pl.ds(s * RPT, RPT)],
                        out_hbm.at[c, pl.ds(s * RPT, RPT)])

    return k(eflat, u1, zeros_z)


def _sc_conv2(eflat, ua, ub, zeros_z):
    eps_ = E // NS
    steps = eps_ // K

    @functools.partial(
        pl.kernel,
        out_type=jax.ShapeDtypeStruct((NC, NPAD, 16), jnp.float32),
        mesh=plsc.VectorSubcoreMesh(**_MESH),
        compiler_params=pltpu.CompilerParams(use_tc_tiling_on_sc=False),
        scratch_types=[
            pltpu.VMEM((K,), jnp.int32),
            pltpu.VMEM((K,), jnp.int32),
            pltpu.VMEM((K, 16), jnp.float32),
            pltpu.VMEM_SHARED((NPAD, 16), jnp.float32),
            pltpu.SemaphoreType.DMA,
        ],
    )
    def k(edge_hbm, ua_hbm, ub_hbm, z_hbm, out_hbm, src_v, dst_v, rows_v, acc, sem):
        c = lax.axis_index("c")
        s = lax.axis_index("s")
        for j in range(RPT // ZCH):
            pltpu.sync_copy(z_hbm, acc.at[pl.ds(s * RPT + j * ZCH, ZCH)])
        plsc.subcore_barrier()
        ebase = s * eps_

        def run(u_hbm):
            def body(i, carry):
                base = ebase + i * K
                pltpu.sync_copy(edge_hbm.at[pl.ds(base, K)], src_v)
                pltpu.sync_copy(edge_hbm.at[pl.ds(E + base, K)], dst_v)
                pltpu.async_copy(u_hbm.at[src_v], rows_v, sem).wait()
                pltpu.sync_copy(rows_v, acc.at[dst_v], add=True)
                return carry
            lax.fori_loop(0, steps, body, 0)

        @pl.when(c == 0)
        def _():
            run(ua_hbm)

        @pl.when(c == 1)
        def _():
            run(ub_hbm)

        plsc.subcore_barrier()
        pltpu.sync_copy(acc.at[pl.ds(s * RPT, RPT)],
                        out_hbm.at[c, pl.ds(s * RPT, RPT)])

    return k(eflat, ua, ub, zeros_z)


def _pk_spec():
    return pl.BlockSpec((BP, 128), lambda i: (i, 0))


def _part_spec(core):
    return pl.BlockSpec((1, BP, 128), lambda i, _c=core: (_c, i, 0))


def _full_spec(shape):
    return pl.BlockSpec(shape, lambda i: tuple(0 for _ in shape))


def _silu(z):
    return z * jax.nn.sigmoid(z)


def _dot(a, b):
    return jnp.dot(a, b, preferred_element_type=jnp.float32)


def _stats(pid, h, S_ref, Q_ref):
    @pl.when(pid == 0)
    def _():
        S_ref[...] = jnp.zeros_like(S_ref)
        Q_ref[...] = jnp.zeros_like(Q_ref)

    row = pid * BP + lax.broadcasted_iota(jnp.int32, (BP, 128), 0)
    hm = jnp.where(row < PROW, h, 0.0)
    S_ref[...] += jnp.sum(hm, axis=0, keepdims=True)
    Q_ref[...] += jnp.sum(hm * hm, axis=0, keepdims=True)


def _gnorm(h, S, Q, M, w, b, a):
    m = _dot(S, M) * (1.0 / N)
    q = _dot(Q, M) * (1.0 / N)
    var = q - (2.0 * a - a * a) * m * m
    return w * (h - a * m) * lax.rsqrt(var + 1e-5) + b


def _tc_a(x16p, degp):
    def body(x_ref, da_ref, db_ref, u1_ref, dinv_ref):
        dinv = lax.rsqrt(da_ref[0] + db_ref[0] + 1.0)
        dinv_ref[...] = dinv
        u1_ref[...] = x_ref[...] * dinv

    return pl.pallas_call(
        body,
        grid=(GRID,),
        in_specs=[_pk_spec(), _part_spec(0), _part_spec(1)],
        out_specs=[_pk_spec(), _pk_spec()],
        out_shape=[jax.ShapeDtypeStruct((PD, 128), jnp.float32),
                   jax.ShapeDtypeStruct((PD, 128), jnp.float32)],
    )(x16p, degp, degp)


def _tc_b(t1v, u1p, dinvp, W1s, B1):
    def body(t1a_ref, t1b_ref, u1_ref, dinv_ref, W_ref, B_ref,
             ha_ref, hb_ref, Sa_ref, Qa_ref, Sb_ref, Qb_ref):
        pid = pl.program_id(0)
        agg = dinv_ref[...] * (t1a_ref[0] + t1b_ref[0] + u1_ref[...])
        for j, (h_ref, S_ref, Q_ref) in enumerate(
                [(ha_ref, Sa_ref, Qa_ref), (hb_ref, Sb_ref, Qb_ref)]):
            h = _silu(_dot(agg, W_ref[j]) + B_ref[j:j + 1, :])
            h_ref[...] = h
            _stats(pid, h, S_ref, Q_ref)

    st = jax.ShapeDtypeStruct((1, 128), jnp.float32)
    pk = jax.ShapeDtypeStruct((PD, 128), jnp.float32)
    return pl.pallas_call(
        body,
        grid=(GRID,),
        in_specs=[_part_spec(0), _part_spec(1), _pk_spec(), _pk_spec(),
                  _full_spec((2, 128, 128)), _full_spec((2, 128))],
        out_specs=[_pk_spec(), _pk_spec(), _full_spec((1, 128)),
                   _full_spec((1, 128)), _full_spec((1, 128)),
                   _full_spec((1, 128))],
        out_shape=[pk, pk, st, st, st, st],
    )(t1v, t1v, u1p, dinvp, W1s, B1)


def _tc_c(h1a, h1b, Sa, Qa, Sb, Qb, dinvp, M, G1w, G1b, G1a):
    def body(ha_ref, hb_ref, Sa_ref, Qa_ref, Sb_ref, Qb_ref, dinv_ref,
             M_ref, Gw_ref, Gb_ref, Ga_ref, ua_ref, ub_ref):
        dinv = dinv_ref[...]
        Mm = M_ref[...]
        for j, (h_ref, S_ref, Q_ref, u_ref) in enumerate(
                [(ha_ref, Sa_ref, Qa_ref, ua_ref), (hb_ref, Sb_ref, Qb_ref, ub_ref)]):
            g = _gnorm(h_ref[...], S_ref[...], Q_ref[...], Mm,
                       Gw_ref[j:j + 1, :], Gb_ref[j:j + 1, :], Ga_ref[j:j + 1, :])
            u_ref[...] = dinv * g

    pk = jax.ShapeDtypeStruct((PD, 128), jnp.float32)
    return pl.pallas_call(
        body,
        grid=(GRID,),
        in_specs=[_pk_spec(), _pk_spec(),
                  _full_spec((1, 128)), _full_spec((1, 128)),
                  _full_spec((1, 128)), _full_spec((1, 128)),
                  _pk_spec(), _full_spec((128, 128)),
                  _full_spec((2, 128)), _full_spec((2, 128)), _full_spec((2, 128))],
        out_specs=[_pk_spec(), _pk_spec()],
        out_shape=[pk, pk],
    )(h1a, h1b, Sa, Qa, Sb, Qb, dinvp, M, G1w, G1b, G1a)


def _tc_d(t2v, uap, ubp, dinvp, W2s, B2):
    def body(t2a_ref, t2b_ref, ua_ref, ub_ref, dinv_ref, W_ref, B_ref,
             h0_ref, h1_ref, h2_ref, h3_ref,
             S0_ref, Q0_ref, S1_ref, Q1_ref, S2_ref, Q2_ref, S3_ref, Q3_ref):
        pid = pl.program_id(0)
        dinv = dinv_ref[...]
        agga = dinv * (t2a_ref[0] + ua_ref[...])
        aggb = dinv * (t2b_ref[0] + ub_ref[...])
        outs = [(h0_ref, S0_ref, Q0_ref), (h1_ref, S1_ref, Q1_ref),
                (h2_ref, S2_ref, Q2_ref), (h3_ref, S3_ref, Q3_ref)]
        for j, (h_ref, S_ref, Q_ref) in enumerate(outs):
            z = _dot(agga, W_ref[j]) + _dot(aggb, W_ref[4 + j]) + B_ref[j:j + 1, :]
            h = _silu(z)
            h_ref[...] = h
            _stats(pid, h, S_ref, Q_ref)

    st = jax.ShapeDtypeStruct((1, 128), jnp.float32)
    pk = jax.ShapeDtypeStruct((PD, 128), jnp.float32)
    return pl.pallas_call(
        body,
        grid=(GRID,),
        in_specs=[_part_spec(0), _part_spec(1), _pk_spec(), _pk_spec(),
                  _pk_spec(), _full_spec((8, 128, 128)), _full_spec((4, 128))],
        out_specs=[_pk_spec()] * 4 + [_full_spec((1, 128))] * 8,
        out_shape=[pk] * 4 + [st] * 8,
    )(t2v, t2v, uap, ubp, dinvp, W2s, B2)


def _tc_e(h2q, S2, Q2, M, G2w, G2b, G2a, L1s, BL1, L2s, BL2, L3s, BL3, L4, BL4):
    def body(h0_ref, h1_ref, h2_ref, h3_ref, S_ref, Q_ref, M_ref,
             Gw_ref, Gb_ref, Ga_ref, L1_ref, BL1_ref, L2_ref, BL2_ref,
             L3_ref, BL3_ref, L4_ref, BL4_ref, out_ref):
        Mm = M_ref[...]
        hrefs = [h0_ref, h1_ref, h2_ref, h3_ref]
        g = [_gnorm(hrefs[q][...], S_ref[q:q + 1, :], Q_ref[q:q + 1, :], Mm,
                    Gw_ref[q:q + 1, :], Gb_ref[q:q + 1, :], Ga_ref[q:q + 1, :])
             for q in range(4)]
        y1 = []
        for j in range(4):
            z = BL1_ref[j:j + 1, :]
            for q in range(4):
                z = z + _dot(g[q], L1_ref[q * 4 + j])
            y1.append(_silu(z))
        y2 = []
        for h in range(2):
            z = BL2_ref[h:h + 1, :]
            for j in range(4):
                z = z + _dot(y1[j], L2_ref[j * 2 + h])
            y2.append(_silu(z))
        y3 = _silu(_dot(y2[0], L3_ref[0]) + _dot(y2[1], L3_ref[1]) + BL3_ref[...])
        out_ref[...] = _dot(y3, L4_ref[...]) + BL4_ref[...]

    return pl.pallas_call(
        body,
        grid=(GRID,),
        in_specs=[_pk_spec()] * 4 +
                 [_full_spec((4, 128)), _full_spec((4, 128)),
                  _full_spec((128, 128)),
                  _full_spec((4, 128)), _full_spec((4, 128)), _full_spec((4, 128)),
                  _full_spec((16, 128, 128)), _full_spec((4, 128)),
                  _full_spec((8, 128, 128)), _full_spec((2, 128)),
                  _full_spec((2, 128, 128)), _full_spec((1, 128)),
                  _full_spec((128, 32)), _full_spec((1, 32))],
        out_specs=[pl.BlockSpec((BP, 32), lambda i: (i, 0))],
        out_shape=[jax.ShapeDtypeStruct((PD, 32), jnp.float32)],
    )(*h2q, S2, Q2, M, G2w, G2b, G2a, L1s, BL1, L2s, BL2, L3s, BL3, L4, BL4)[0]


def kernel(x, edge_index, W1, b1, gn1_w, gn1_b, gn1_a, W2, b2, gn2_w, gn2_b,
           gn2_a, l1_w, l1_b, l2_w, l2_b, l3_w, l3_b, l4_w, l4_b):
    f32 = jnp.float32
    eflat = edge_index.reshape(2 * E)
    ones_k = jnp.ones((K, 16), f32)
    zeros_z = jnp.zeros((ZCH, 16), f32)

    ey8 = jnp.eye(8, dtype=f32)

    def bd(w16):
        return jnp.kron(ey8, w16)

    def t8(v16):
        return jnp.tile(v16, 8)

    M = jnp.tile(jnp.eye(16, dtype=f32), (8, 8))

    W1s = jnp.stack([bd(jnp.pad(W1[:, 16 * j:16 * (j + 1)], ((0, 8), (0, 0))))
                     for j in range(2)])
    B1 = jnp.stack([t8(b1[16 * j:16 * (j + 1)]) for j in range(2)])

    W2s = jnp.stack([bd(W2[16 * a:16 * (a + 1), 16 * j:16 * (j + 1)])
                     for a in range(2) for j in range(4)])
    B2 = jnp.stack([t8(b2[16 * j:16 * (j + 1)]) for j in range(4)])

    G1w = jnp.stack([t8(gn1_w[16 * j:16 * (j + 1)]) for j in range(2)])
    G1b = jnp.stack([t8(gn1_b[16 * j:16 * (j + 1)]) for j in range(2)])
    G1a = jnp.stack([t8(gn1_a[16 * j:16 * (j + 1)]) for j in range(2)])
    G2w = jnp.stack([t8(gn2_w[16 * q:16 * (q + 1)]) for q in range(4)])
    G2b = jnp.stack([t8(gn2_b[16 * q:16 * (q + 1)]) for q in range(4)])
    G2a = jnp.stack([t8(gn2_a[16 * q:16 * (q + 1)]) for q in range(4)])

    L1s = jnp.stack([bd(l1_w[16 * q:16 * (q + 1), 16 * j:16 * (j + 1)])
                     for q in range(4) for j in range(4)])
    BL1 = jnp.stack([t8(l1_b[16 * j:16 * (j + 1)]) for j in range(4)])
    L2s = jnp.stack([bd(l2_w[16 * q:16 * (q + 1), 16 * h:16 * (h + 1)])
                     for q in range(4) for h in range(2)])
    BL2 = jnp.stack([t8(l2_b[16 * h:16 * (h + 1)]) for h in range(2)])
    L3s = jnp.stack([bd(l3_w[16 * h:16 * (h + 1), :]) for h in range(2)])
    BL3 = t8(l3_b).reshape(1, 128)
    L4 = jnp.kron(ey8, l4_w)
    BL4 = jnp.tile(l4_b, 8).reshape(1, 32)

    x16p = jnp.pad(x, ((0, NPAD - N), (0, 8))).reshape(PD, 128)

    degp = _sc_degree(eflat, zeros_z, ones_k)
    u1p, dinvp = _tc_a(x16p, degp.reshape(NC, PD, 128))

    t1p = _sc_conv1(eflat, u1p.reshape(NPAD, 16), zeros_z)
    h1a, h1b, Sa, Qa, Sb, Qb = _tc_b(t1p.reshape(NC, PD, 128), u1p, dinvp,
                                     W1s, B1)

    uap, ubp = _tc_c(h1a, h1b, Sa, Qa, Sb, Qb, dinvp, M, G1w, G1b, G1a)

    t2p = _sc_conv2(eflat, uap.reshape(NPAD, 16), ubp.reshape(NPAD, 16), zeros_z)
    dres = _tc_d(t2p.reshape(NC, PD, 128), uap, ubp, dinvp, W2s, B2)
    h2q, stats = dres[:4], dres[4:]
    S2 = jnp.concatenate([stats[0], stats[2], stats[4], stats[6]], axis=0)
    Q2 = jnp.concatenate([stats[1], stats[3], stats[5], stats[7]], axis=0)

    outp = _tc_e(h2q, S2, Q2, M, G2w, G2b, G2a, L1s, BL1, L2s, BL2,
                 L3s, BL3, L4, BL4)
    return outp[:PROW].reshape(N, 4)


# speedup vs baseline: 60.1839x; 1.3610x over previous
import functools

import jax
import jax.numpy as jnp
from jax import lax
from jax.experimental import pallas as pl
from jax.experimental.pallas import tpu as pltpu
from jax.experimental.pallas import tpu_sc as plsc

N = 100000
E = 1600000
NC = 2
NS = 16
NW = NC * NS
RPT = 6400
NPAD = NS * RPT
ZCH = 1600
K = 1000

PD = NPAD * 16 // 128
PROW = N * 16 // 128
BP = 512
GRID = PD // BP

_MESH = dict(core_axis_name="c", subcore_axis_name="s")


KH = 500
SUP = 4
DROW = E // KH
ER = 2 * E // KH


def _edge_pipeline(e_hbm, u_hbm, acc, sAs, sAd, sBs, sBd, R0, R1,
                   isem, g0, g1, rbase, T):
    def idx_issue(bs, bd, t):
        r = rbase + t * SUP
        pltpu.async_copy(e_hbm.at[pl.ds(r, SUP)], bs, isem)
        pltpu.async_copy(e_hbm.at[pl.ds(DROW + r, SUP)], bd, isem)

    def idx_wait(bs, bd, t):
        r = rbase + t * SUP
        pltpu.make_async_copy(e_hbm.at[pl.ds(r, SUP)], bs, isem).wait()
        pltpu.make_async_copy(e_hbm.at[pl.ds(DROW + r, SUP)], bd, isem).wait()

    def g_issue(bs, j, R, sem):
        pltpu.async_copy(u_hbm.at[bs.at[j]], R, sem)

    def g_wait(bs, j, R, sem):
        pltpu.make_async_copy(u_hbm.at[bs.at[j]], R, sem).wait()

    pltpu.sync_copy(e_hbm.at[pl.ds(rbase, SUP)], sAs)
    pltpu.sync_copy(e_hbm.at[pl.ds(DROW + rbase, SUP)], sAd)
    idx_issue(sBs, sBd, 1)
    g_issue(sAs, 0, R0, g0)

    def super_body(cur_s, cur_d, nxt_s, nxt_d, t):
        bufs = [(R0, g0), (R1, g1)]
        for j in range(SUP):
            Rc, semc = bufs[j % 2]
            Rn, semn = bufs[(j + 1) % 2]
            if j < SUP - 1:
                g_issue(cur_s, j + 1, Rn, semn)
            else:
                @pl.when(t + 1 < T)
                def _():
                    idx_wait(nxt_s, nxt_d, t + 1)
                    g_issue(nxt_s, 0, R0, g0)
            g_wait(cur_s, j, Rc, semc)
            pltpu.sync_copy(Rc, acc.at[cur_d.at[j]], add=True)

        @pl.when(t + 2 < T)
        def _():
            idx_issue(cur_s, cur_d, t + 2)

    def body(t, carry):
        @pl.when(t % 2 == 0)
        def _():
            super_body(sAs, sAd, sBs, sBd, t)

        @pl.when(t % 2 == 1)
        def _():
            super_body(sBs, sBd, sAs, sAd, t)

        return carry

    lax.fori_loop(0, T, body, 0)


def _zero_acc(z_hbm, acc, s):
    for j in range(RPT // ZCH):
        pltpu.sync_copy(z_hbm, acc.at[pl.ds(s * RPT + j * ZCH, ZCH)])


def _writeback(acc, out_hbm, c, s):
    pltpu.sync_copy(acc.at[pl.ds(s * RPT, RPT)],
                    out_hbm.at[c, pl.ds(s * RPT, RPT)])


def _sc_degree(eflat2, zeros_z, ones_k):
    T = E // NW // (SUP * KH)

    @functools.partial(
        pl.kernel,
        out_type=jax.ShapeDtypeStruct((NC, NPAD, 16), jnp.float32),
        mesh=plsc.VectorSubcoreMesh(**_MESH),
        compiler_params=pltpu.CompilerParams(use_tc_tiling_on_sc=False),
        scratch_types=[
            pltpu.VMEM((SUP, KH), jnp.int32),
            pltpu.VMEM((SUP, KH), jnp.int32),
            pltpu.VMEM((KH, 16), jnp.float32),
            pltpu.VMEM_SHARED((NPAD, 16), jnp.float32),
            pltpu.SemaphoreType.DMA,
        ],
    )
    def k(e_hbm, z_hbm, ones_hbm, out_hbm, dA, dB, ones_v, acc, isem):
        c = lax.axis_index("c")
        s = lax.axis_index("s")
        w = s * NC + c
        rbase = w * (E // NW // KH)

        pltpu.sync_copy(ones_hbm, ones_v)
        _zero_acc(z_hbm, acc, s)
        plsc.subcore_barrier()

        def idx_issue(bd, t):
            pltpu.async_copy(e_hbm.at[pl.ds(DROW + rbase + t * SUP, SUP)],
                             bd, isem)

        def idx_wait(bd, t):
            pltpu.make_async_copy(e_hbm.at[pl.ds(DROW + rbase + t * SUP, SUP)],
                                  bd, isem).wait()

        pltpu.sync_copy(e_hbm.at[pl.ds(DROW + rbase, SUP)], dA)
        idx_issue(dB, 1)

        def super_body(cur_d, nxt_d, t):
            for j in range(SUP):
                pltpu.sync_copy(ones_v, acc.at[cur_d.at[j]], add=True)

            @pl.when(t + 1 < T)
            def _():
                idx_wait(nxt_d, t + 1)

            @pl.when(t + 2 < T)
            def _():
                idx_issue(cur_d, t + 2)

        def body(t, carry):
            @pl.when(t % 2 == 0)
            def _():
                super_body(dA, dB, t)

            @pl.when(t % 2 == 1)
            def _():
                super_body(dB, dA, t)

            return carry

        lax.fori_loop(0, T, body, 0)
        plsc.subcore_barrier()
        _writeback(acc, out_hbm, c, s)

    return k(eflat2, zeros_z, ones_k)


def _sc_conv1(eflat2, u1, zeros_z):
    T = E // NW // (SUP * KH)

    @functools.partial(
        pl.kernel,
        out_type=jax.ShapeDtypeStruct((NC, NPAD, 16), jnp.float32),
        mesh=plsc.VectorSubcoreMesh(**_MESH),
        compiler_params=pltpu.CompilerParams(use_tc_tiling_on_sc=False),
        scratch_types=[
            pltpu.VMEM((SUP, KH), jnp.int32),
            pltpu.VMEM((SUP, KH), jnp.int32),
            pltpu.VMEM((SUP, KH), jnp.int32),
            pltpu.VMEM((SUP, KH), jnp.int32),
            pltpu.VMEM((KH, 16), jnp.float32),
            pltpu.VMEM((KH, 16), jnp.float32),
            pltpu.VMEM_SHARED((NPAD, 16), jnp.float32),
            pltpu.SemaphoreType.DMA,
            pltpu.SemaphoreType.DMA,
            pltpu.SemaphoreType.DMA,
        ],
    )
    def k(e_hbm, u_hbm, z_hbm, out_hbm, sAs, sAd, sBs, sBd, R0, R1, acc,
          isem, g0, g1):
        c = lax.axis_index("c")
        s = lax.axis_index("s")
        w = s * NC + c
        _zero_acc(z_hbm, acc, s)
        plsc.subcore_barrier()
        _edge_pipeline(e_hbm, u_hbm, acc, sAs, sAd, sBs, sBd, R0, R1,
                       isem, g0, g1, w * (E // NW // KH), T)
        plsc.subcore_barrier()
        _writeback(acc, out_hbm, c, s)

    return k(eflat2, u1, zeros_z)


def _sc_conv2(eflat2, ua, ub, zeros_z):
    T = E // NS // (SUP * KH)

    @functools.partial(
        pl.kernel,
        out_type=jax.ShapeDtypeStruct((NC, NPAD, 16), jnp.float32),
        mesh=plsc.VectorSubcoreMesh(**_MESH),
        compiler_params=pltpu.CompilerParams(use_tc_tiling_on_sc=False),
        scratch_types=[
            pltpu.VMEM((SUP, KH), jnp.int32),
            pltpu.VMEM((SUP, KH), jnp.int32),
            pltpu.VMEM((SUP, KH), jnp.int32),
            pltpu.VMEM((SUP, KH), jnp.int32),
            pltpu.VMEM((KH, 16), jnp.float32),
            pltpu.VMEM((KH, 16), jnp.float32),
            pltpu.VMEM_SHARED((NPAD, 16), jnp.float32),
            pltpu.SemaphoreType.DMA,
            pltpu.SemaphoreType.DMA,
            pltpu.SemaphoreType.DMA,
        ],
    )
    def k(e_hbm, ua_hbm, ub_hbm, z_hbm, out_hbm, sAs, sAd, sBs, sBd, R0, R1,
          acc, isem, g0, g1):
        c = lax.axis_index("c")
        s = lax.axis_index("s")
        _zero_acc(z_hbm, acc, s)
        plsc.subcore_barrier()
        rbase = s * (E // NS // KH)

        @pl.when(c == 0)
        def _():
            _edge_pipeline(e_hbm, ua_hbm, acc, sAs, sAd, sBs, sBd, R0, R1,
                           isem, g0, g1, rbase, T)

        @pl.when(c == 1)
        def _():
            _edge_pipeline(e_hbm, ub_hbm, acc, sAs, sAd, sBs, sBd, R0, R1,
                           isem, g0, g1, rbase, T)

        plsc.subcore_barrier()
        _writeback(acc, out_hbm, c, s)

    return k(eflat2, ua, ub, zeros_z)


def _pk_spec():
    return pl.BlockSpec((BP, 128), lambda i: (i, 0))


def _part_spec(core):
    return pl.BlockSpec((1, BP, 128), lambda i, _c=core: (_c, i, 0))


def _full_spec(shape):
    return pl.BlockSpec(shape, lambda i: tuple(0 for _ in shape))


def _silu(z):
    return z * jax.nn.sigmoid(z)


def _dot(a, b):
    return jnp.dot(a, b, preferred_element_type=jnp.float32)


def _stats(pid, h, S_ref, Q_ref):
    @pl.when(pid == 0)
    def _():
        S_ref[...] = jnp.zeros_like(S_ref)
        Q_ref[...] = jnp.zeros_like(Q_ref)

    row = pid * BP + lax.broadcasted_iota(jnp.int32, (BP, 128), 0)
    hm = jnp.where(row < PROW, h, 0.0)
    S_ref[...] += jnp.sum(hm, axis=0, keepdims=True)
    Q_ref[...] += jnp.sum(hm * hm, axis=0, keepdims=True)


def _gnorm(h, S, Q, M, w, b, a):
    m = _dot(S, M) * (1.0 / N)
    q = _dot(Q, M) * (1.0 / N)
    var = q - (2.0 * a - a * a) * m * m
    return w * (h - a * m) * lax.rsqrt(var + 1e-5) + b


def _tc_a(x16p, degp):
    def body(x_ref, da_ref, db_ref, u1_ref, dinv_ref):
        dinv = lax.rsqrt(da_ref[0] + db_ref[0] + 1.0)
        dinv_ref[...] = dinv
        u1_ref[...] = x_ref[...] * dinv

    return pl.pallas_call(
        body,
        grid=(GRID,),
        in_specs=[_pk_spec(), _part_spec(0), _part_spec(1)],
        out_specs=[_pk_spec(), _pk_spec()],
        out_shape=[jax.ShapeDtypeStruct((PD, 128), jnp.float32),
                   jax.ShapeDtypeStruct((PD, 128), jnp.float32)],
    )(x16p, degp, degp)


def _tc_b(t1v, u1p, dinvp, W1s, B1):
    def body(t1a_ref, t1b_ref, u1_ref, dinv_ref, W_ref, B_ref,
             ha_ref, hb_ref, Sa_ref, Qa_ref, Sb_ref, Qb_ref):
        pid = pl.program_id(0)
        agg = dinv_ref[...] * (t1a_ref[0] + t1b_ref[0] + u1_ref[...])
        for j, (h_ref, S_ref, Q_ref) in enumerate(
                [(ha_ref, Sa_ref, Qa_ref), (hb_ref, Sb_ref, Qb_ref)]):
            h = _silu(_dot(agg, W_ref[j]) + B_ref[j:j + 1, :])
            h_ref[...] = h
            _stats(pid, h, S_ref, Q_ref)

    st = jax.ShapeDtypeStruct((1, 128), jnp.float32)
    pk = jax.ShapeDtypeStruct((PD, 128), jnp.float32)
    return pl.pallas_call(
        body,
        grid=(GRID,),
        in_specs=[_part_spec(0), _part_spec(1), _pk_spec(), _pk_spec(),
                  _full_spec((2, 128, 128)), _full_spec((2, 128))],
        out_specs=[_pk_spec(), _pk_spec(), _full_spec((1, 128)),
                   _full_spec((1, 128)), _full_spec((1, 128)),
                   _full_spec((1, 128))],
        out_shape=[pk, pk, st, st, st, st],
    )(t1v, t1v, u1p, dinvp, W1s, B1)


def _tc_c(h1a, h1b, Sa, Qa, Sb, Qb, dinvp, M, G1w, G1b, G1a):
    def body(ha_ref, hb_ref, Sa_ref, Qa_ref, Sb_ref, Qb_ref, dinv_ref,
             M_ref, Gw_ref, Gb_ref, Ga_ref, ua_ref, ub_ref):
        dinv = dinv_ref[...]
        Mm = M_ref[...]
        for j, (h_ref, S_ref, Q_ref, u_ref) in enumerate(
                [(ha_ref, Sa_ref, Qa_ref, ua_ref), (hb_ref, Sb_ref, Qb_ref, ub_ref)]):
            g = _gnorm(h_ref[...], S_ref[...], Q_ref[...], Mm,
                       Gw_ref[j:j + 1, :], Gb_ref[j:j + 1, :], Ga_ref[j:j + 1, :])
            u_ref[...] = dinv * g

    pk = jax.ShapeDtypeStruct((PD, 128), jnp.float32)
    return pl.pallas_call(
        body,
        grid=(GRID,),
        in_specs=[_pk_spec(), _pk_spec(),
                  _full_spec((1, 128)), _full_spec((1, 128)),
                  _full_spec((1, 128)), _full_spec((1, 128)),
                  _pk_spec(), _full_spec((128, 128)),
                  _full_spec((2, 128)), _full_spec((2, 128)), _full_spec((2, 128))],
        out_specs=[_pk_spec(), _pk_spec()],
        out_shape=[pk, pk],
    )(h1a, h1b, Sa, Qa, Sb, Qb, dinvp, M, G1w, G1b, G1a)


def _tc_d(t2v, uap, ubp, dinvp, W2s, B2):
    def body(t2a_ref, t2b_ref, ua_ref, ub_ref, dinv_ref, W_ref, B_ref,
             h0_ref, h1_ref, h2_ref, h3_ref,
             S0_ref, Q0_ref, S1_ref, Q1_ref, S2_ref, Q2_ref, S3_ref, Q3_ref):
        pid = pl.program_id(0)
        dinv = dinv_ref[...]
        agga = dinv * (t2a_ref[0] + ua_ref[...])
        aggb = dinv * (t2b_ref[0] + ub_ref[...])
        outs = [(h0_ref, S0_ref, Q0_ref), (h1_ref, S1_ref, Q1_ref),
                (h2_ref, S2_ref, Q2_ref), (h3_ref, S3_ref, Q3_ref)]
        for j, (h_ref, S_ref, Q_ref) in enumerate(outs):
            z = _dot(agga, W_ref[j]) + _dot(aggb, W_ref[4 + j]) + B_ref[j:j + 1, :]
            h = _silu(z)
            h_ref[...] = h
            _stats(pid, h, S_ref, Q_ref)

    st = jax.ShapeDtypeStruct((1, 128), jnp.float32)
    pk = jax.ShapeDtypeStruct((PD, 128), jnp.float32)
    return pl.pallas_call(
        body,
        grid=(GRID,),
        in_specs=[_part_spec(0), _part_spec(1), _pk_spec(), _pk_spec(),
                  _pk_spec(), _full_spec((8, 128, 128)), _full_spec((4, 128))],
        out_specs=[_pk_spec()] * 4 + [_full_spec((1, 128))] * 8,
        out_shape=[pk] * 4 + [st] * 8,
    )(t2v, t2v, uap, ubp, dinvp, W2s, B2)


def _tc_e(h2q, S2, Q2, M, G2w, G2b, G2a, L1s, BL1, L2s, BL2, L3s, BL3, L4, BL4):
    def body(h0_ref, h1_ref, h2_ref, h3_ref, S_ref, Q_ref, M_ref,
             Gw_ref, Gb_ref, Ga_ref, L1_ref, BL1_ref, L2_ref, BL2_ref,
             L3_ref, BL3_ref, L4_ref, BL4_ref, out_ref):
        Mm = M_ref[...]
        hrefs = [h0_ref, h1_ref, h2_ref, h3_ref]
        g = [_gnorm(hrefs[q][...], S_ref[q:q + 1, :], Q_ref[q:q + 1, :], Mm,
                    Gw_ref[q:q + 1, :], Gb_ref[q:q + 1, :], Ga_ref[q:q + 1, :])
             for q in range(4)]
        y1 = []
        for j in range(4):
            z = BL1_ref[j:j + 1, :]
            for q in range(4):
                z = z + _dot(g[q], L1_ref[q * 4 + j])
            y1.append(_silu(z))
        y2 = []
        for h in range(2):
            z = BL2_ref[h:h + 1, :]
            for j in range(4):
                z = z + _dot(y1[j], L2_ref[j * 2 + h])
            y2.append(_silu(z))
        y3 = _silu(_dot(y2[0], L3_ref[0]) + _dot(y2[1], L3_ref[1]) + BL3_ref[...])
        out_ref[...] = _dot(y3, L4_ref[...]) + BL4_ref[...]

    return pl.pallas_call(
        body,
        grid=(GRID,),
        in_specs=[_pk_spec()] * 4 +
                 [_full_spec((4, 128)), _full_spec((4, 128)),
                  _full_spec((128, 128)),
                  _full_spec((4, 128)), _full_spec((4, 128)), _full_spec((4, 128)),
                  _full_spec((16, 128, 128)), _full_spec((4, 128)),
                  _full_spec((8, 128, 128)), _full_spec((2, 128)),
                  _full_spec((2, 128, 128)), _full_spec((1, 128)),
                  _full_spec((128, 32)), _full_spec((1, 32))],
        out_specs=[pl.BlockSpec((BP, 32), lambda i: (i, 0))],
        out_shape=[jax.ShapeDtypeStruct((PD, 32), jnp.float32)],
    )(*h2q, S2, Q2, M, G2w, G2b, G2a, L1s, BL1, L2s, BL2, L3s, BL3, L4, BL4)[0]


def kernel(x, edge_index, W1, b1, gn1_w, gn1_b, gn1_a, W2, b2, gn2_w, gn2_b,
           gn2_a, l1_w, l1_b, l2_w, l2_b, l3_w, l3_b, l4_w, l4_b):
    f32 = jnp.float32
    eflat2 = edge_index.reshape(ER, KH)
    ones_k = jnp.ones((KH, 16), f32)
    zeros_z = jnp.zeros((ZCH, 16), f32)

    ey8 = jnp.eye(8, dtype=f32)

    def bd(w16):
        return jnp.kron(ey8, w16)

    def t8(v16):
        return jnp.tile(v16, 8)

    M = jnp.tile(jnp.eye(16, dtype=f32), (8, 8))

    W1s = jnp.stack([bd(jnp.pad(W1[:, 16 * j:16 * (j + 1)], ((0, 8), (0, 0))))
                     for j in range(2)])
    B1 = jnp.stack([t8(b1[16 * j:16 * (j + 1)]) for j in range(2)])

    W2s = jnp.stack([bd(W2[16 * a:16 * (a + 1), 16 * j:16 * (j + 1)])
                     for a in range(2) for j in range(4)])
    B2 = jnp.stack([t8(b2[16 * j:16 * (j + 1)]) for j in range(4)])

    G1w = jnp.stack([t8(gn1_w[16 * j:16 * (j + 1)]) for j in range(2)])
    G1b = jnp.stack([t8(gn1_b[16 * j:16 * (j + 1)]) for j in range(2)])
    G1a = jnp.stack([t8(gn1_a[16 * j:16 * (j + 1)]) for j in range(2)])
    G2w = jnp.stack([t8(gn2_w[16 * q:16 * (q + 1)]) for q in range(4)])
    G2b = jnp.stack([t8(gn2_b[16 * q:16 * (q + 1)]) for q in range(4)])
    G2a = jnp.stack([t8(gn2_a[16 * q:16 * (q + 1)]) for q in range(4)])

    L1s = jnp.stack([bd(l1_w[16 * q:16 * (q + 1), 16 * j:16 * (j + 1)])
                     for q in range(4) for j in range(4)])
    BL1 = jnp.stack([t8(l1_b[16 * j:16 * (j + 1)]) for j in range(4)])
    L2s = jnp.stack([bd(l2_w[16 * q:16 * (q + 1), 16 * h:16 * (h + 1)])
                     for q in range(4) for h in range(2)])
    BL2 = jnp.stack([t8(l2_b[16 * h:16 * (h + 1)]) for h in range(2)])
    L3s = jnp.stack([bd(l3_w[16 * h:16 * (h + 1), :]) for h in range(2)])
    BL3 = t8(l3_b).reshape(1, 128)
    L4 = jnp.kron(ey8, l4_w)
    BL4 = jnp.tile(l4_b, 8).reshape(1, 32)

    x16p = jnp.pad(x, ((0, NPAD - N), (0, 8))).reshape(PD, 128)

    degp = _sc_degree(eflat2, zeros_z, ones_k)
    u1p, dinvp = _tc_a(x16p, degp.reshape(NC, PD, 128))

    t1p = _sc_conv1(eflat2, u1p.reshape(NPAD, 16), zeros_z)
    h1a, h1b, Sa, Qa, Sb, Qb = _tc_b(t1p.reshape(NC, PD, 128), u1p, dinvp,
                                     W1s, B1)

    uap, ubp = _tc_c(h1a, h1b, Sa, Qa, Sb, Qb, dinvp, M, G1w, G1b, G1a)

    t2p = _sc_conv2(eflat2, uap.reshape(NPAD, 16), ubp.reshape(NPAD, 16), zeros_z)
    dres = _tc_d(t2p.reshape(NC, PD, 128), uap, ubp, dinvp, W2s, B2)
    h2q, stats = dres[:4], dres[4:]
    S2 = jnp.concatenate([stats[0], stats[2], stats[4], stats[6]], axis=0)
    Q2 = jnp.concatenate([stats[1], stats[3], stats[5], stats[7]], axis=0)

    outp = _tc_e(h2q, S2, Q2, M, G2w, G2b, G2a, L1s, BL1, L2s, BL2,
                 L3s, BL3, L4, BL4)
    return outp[:PROW].reshape(N, 4)
